# baseline (speedup 1.0000x reference)
import functools

import jax
from jax import lax
import jax.numpy as jnp
from jax.experimental import pallas as pl
from jax.experimental.pallas import tpu as pltpu
from jax.experimental.pallas import tpu_sc as plsc

_NC = 2
_NS = 16
_NW = _NC * _NS
_LANES = 16
_CHUNK = 400


def _sc_gather_body(spd0_hbm, spd1_hbm, tab2_hbm, pe_hbm, idx0_v,
                    idx1_v, pair_v, rows_v, sem, *, num_chunks, table_rows):
    cid = lax.axis_index("c")
    sid = lax.axis_index("s")
    wid = sid * _NC + cid

    base_c, extra = divmod(num_chunks, _NW)
    start = wid * base_c + jnp.minimum(wid, extra)
    count = base_c + jnp.where(wid < extra, 1, 0)

    def chunk_body(ci, carry):
        row0 = ci * _CHUNK
        pltpu.sync_copy(spd0_hbm.at[pl.ds(row0, _CHUNK)], idx0_v)
        pltpu.sync_copy(spd1_hbm.at[pl.ds(row0, _CHUNK)], idx1_v)

        def g(k, c2):
            a = idx0_v[pl.ds(k * _LANES, _LANES)]
            b = idx1_v[pl.ds(k * _LANES, _LANES)]
            pair_v[pl.ds(k * _LANES, _LANES)] = a * table_rows + b
            return c2

        lax.fori_loop(0, _CHUNK // _LANES, g, 0)
        off = 0
        while off < _CHUNK:
            sz = min(128, _CHUNK - off)
            pltpu.async_copy(tab2_hbm.at[pair_v.at[pl.ds(off, sz)]],
                             rows_v.at[pl.ds(off, sz)], sem)
            off += sz
        pltpu.make_async_copy(pe_hbm.at[pl.ds(row0, _CHUNK)], rows_v, sem).wait()
        pltpu.sync_copy(rows_v, pe_hbm.at[pl.ds(row0, _CHUNK)])
        return carry

    lax.fori_loop(start, start + count, chunk_body, 0)


def _sc_gather(spd0, spd1, table2, n):
    table_rows = int(round(table2.shape[0] ** 0.5))
    dim_pe = table2.shape[1]
    num_chunks = n // _CHUNK
    mesh = plsc.VectorSubcoreMesh(core_axis_name="c", subcore_axis_name="s")
    k = functools.partial(
        pl.kernel,
        mesh=mesh,
        out_type=jax.ShapeDtypeStruct((n, dim_pe), jnp.float32),
        scratch_types=[
            pltpu.VMEM((_CHUNK,), jnp.int32),
            pltpu.VMEM((_CHUNK,), jnp.int32),
            pltpu.VMEM((_CHUNK,), jnp.int32),
            pltpu.VMEM((_CHUNK, dim_pe), jnp.float32),
            pltpu.SemaphoreType.DMA,
        ],
        compiler_params=pltpu.CompilerParams(use_tc_tiling_on_sc=False),
    )(functools.partial(_sc_gather_body, num_chunks=num_chunks,
                        table_rows=table_rows))
    return k(spd0, spd1, table2)


def _pick_block(n):
    for blk in (2000, 1000, 500, 200, 100, 40, 8):
        if n % blk == 0:
            return blk
    return n


def _tc_body(x_ref, pe_ref, w_ref, b_ref, out_ref):
    h = jnp.dot(x_ref[...], w_ref[...], preferred_element_type=jnp.float32)
    out_ref[...] = jnp.concatenate([h + b_ref[...], pe_ref[...]], axis=1)


def kernel(x, spd, dist_table, W, b):
    n, dim_in = x.shape
    dim_h = W.shape[1]
    rows, half_pe = dist_table.shape
    dim_pe = 2 * half_pe
    dim_out = dim_h + dim_pe

    table2 = jnp.concatenate(
        [jnp.repeat(dist_table, rows, axis=0), jnp.tile(dist_table, (rows, 1))],
        axis=1)
    spd0 = spd[:, 0]
    spd1 = spd[:, 1]

    pe = _sc_gather(spd0, spd1, table2, n)

    blk = _pick_block(n)
    grid = (n // blk,)
    b2 = b.reshape(1, dim_h)
    return pl.pallas_call(
        _tc_body,
        grid=grid,
        in_specs=[
            pl.BlockSpec((blk, dim_in), lambda i: (i, 0)),
            pl.BlockSpec((blk, dim_pe), lambda i: (i, 0)),
            pl.BlockSpec((dim_in, dim_h), lambda i: (0, 0)),
            pl.BlockSpec((1, dim_h), lambda i: (0, 0)),
        ],
        out_specs=pl.BlockSpec((blk, dim_out), lambda i: (i, 0)),
        out_shape=jax.ShapeDtypeStruct((n, dim_out), jnp.float32),
        compiler_params=pltpu.CompilerParams(
            dimension_semantics=("parallel",),
        ),
    )(x, pe, W, b2)


# speedup vs baseline: 1.0215x; 1.0215x over previous
import functools

import jax
from jax import lax
import jax.numpy as jnp
from jax.experimental import pallas as pl
from jax.experimental.pallas import tpu as pltpu
from jax.experimental.pallas import tpu_sc as plsc

_NC = 2
_NS = 16
_NW = _NC * _NS
_LANES = 16
_CHUNK = 800


def _sc_gather_body(spd0_hbm, spd1_hbm, tab2_hbm, pe_hbm, idx0_v, idx1_v,
                    pair_v, rows_v, sem_idx, sem_g, sem_st, *,
                    num_chunks, table_rows):
    cid = lax.axis_index("c")
    sid = lax.axis_index("s")
    wid = sid * _NC + cid

    base_c, extra = divmod(num_chunks, _NW)
    start = wid * base_c + jnp.minimum(wid, extra)
    count = base_c + jnp.where(wid < extra, 1, 0)

    def fire_idx(ci, s):
        row0 = ci * _CHUNK
        pltpu.make_async_copy(spd0_hbm.at[pl.ds(row0, _CHUNK)],
                              idx0_v.at[s], sem_idx.at[s]).start()
        pltpu.make_async_copy(spd1_hbm.at[pl.ds(row0, _CHUNK)],
                              idx1_v.at[s], sem_idx.at[s]).start()

    def wait_idx(s):
        pltpu.make_async_copy(spd0_hbm.at[pl.ds(0, _CHUNK)],
                              idx0_v.at[s], sem_idx.at[s]).wait()
        pltpu.make_async_copy(spd1_hbm.at[pl.ds(0, _CHUNK)],
                              idx1_v.at[s], sem_idx.at[s]).wait()

    def chunk_body(ci, carry):
        r = ci - start
        s = lax.rem(r, 2)
        row0 = ci * _CHUNK

        @pl.when(r == 0)
        def _():
            fire_idx(ci, s)

        wait_idx(s)

        def g(k, c2):
            a = idx0_v.at[s][pl.ds(k * _LANES, _LANES)]
            b = idx1_v.at[s][pl.ds(k * _LANES, _LANES)]
            pair_v[pl.ds(k * _LANES, _LANES)] = a * table_rows + b
            return c2

        lax.fori_loop(0, _CHUNK // _LANES, g, 0)

        @pl.when(r + 1 < count)
        def _():
            fire_idx(ci + 1, 1 - s)

        @pl.when(r >= 2)
        def _():
            pltpu.make_async_copy(rows_v.at[s],
                                  pe_hbm.at[pl.ds(row0, _CHUNK)],
                                  sem_st.at[s]).wait()

        off = 0
        while off < _CHUNK:
            sz = min(128, _CHUNK - off)
            pltpu.async_copy(tab2_hbm.at[pair_v.at[pl.ds(off, sz)]],
                             rows_v.at[s, pl.ds(off, sz)], sem_g)
            off += sz
        pltpu.make_async_copy(pe_hbm.at[pl.ds(row0, _CHUNK)],
                              rows_v.at[s], sem_g).wait()
        pltpu.make_async_copy(rows_v.at[s],
                              pe_hbm.at[pl.ds(row0, _CHUNK)],
                              sem_st.at[s]).start()
        return carry

    lax.fori_loop(start, start + count, chunk_body, 0, unroll=False)

    @pl.when(count >= 2)
    def _():
        s = lax.rem(count - 2, 2)
        pltpu.make_async_copy(rows_v.at[s], pe_hbm.at[pl.ds(0, _CHUNK)],
                              sem_st.at[s]).wait()

    @pl.when(count >= 1)
    def _():
        s = lax.rem(count - 1, 2)
        pltpu.make_async_copy(rows_v.at[s], pe_hbm.at[pl.ds(0, _CHUNK)],
                              sem_st.at[s]).wait()


def _sc_gather(spd0, spd1, table2, n):
    table_rows = int(round(table2.shape[0] ** 0.5))
    dim_pe = table2.shape[1]
    num_chunks = n // _CHUNK
    mesh = plsc.VectorSubcoreMesh(core_axis_name="c", subcore_axis_name="s")
    k = functools.partial(
        pl.kernel,
        mesh=mesh,
        out_type=jax.ShapeDtypeStruct((n, dim_pe), jnp.float32),
        scratch_types=[
            pltpu.VMEM((2, _CHUNK), jnp.int32),
            pltpu.VMEM((2, _CHUNK), jnp.int32),
            pltpu.VMEM((_CHUNK,), jnp.int32),
            pltpu.VMEM((2, _CHUNK, dim_pe), jnp.float32),
            pltpu.SemaphoreType.DMA((2,)),
            pltpu.SemaphoreType.DMA,
            pltpu.SemaphoreType.DMA((2,)),
        ],
        compiler_params=pltpu.CompilerParams(use_tc_tiling_on_sc=False),
    )(functools.partial(_sc_gather_body, num_chunks=num_chunks,
                        table_rows=table_rows))
    return k(spd0, spd1, table2)


def _pick_block(n):
    for blk in (2000, 1000, 500, 200, 100, 40, 8):
        if n % blk == 0:
            return blk
    return n


def _tc_body(x_ref, pe_ref, w_ref, b_ref, out_ref):
    h = jnp.dot(x_ref[...], w_ref[...], preferred_element_type=jnp.float32)
    out_ref[...] = jnp.concatenate([h + b_ref[...], pe_ref[...]], axis=1)


def kernel(x, spd, dist_table, W, b):
    n, dim_in = x.shape
    dim_h = W.shape[1]
    rows, half_pe = dist_table.shape
    dim_pe = 2 * half_pe
    dim_out = dim_h + dim_pe

    table2 = jnp.concatenate(
        [jnp.repeat(dist_table, rows, axis=0), jnp.tile(dist_table, (rows, 1))],
        axis=1)
    spd0 = spd[:, 0]
    spd1 = spd[:, 1]

    pe = _sc_gather(spd0, spd1, table2, n)

    blk = _pick_block(n)
    grid = (n // blk,)
    b2 = b.reshape(1, dim_h)
    return pl.pallas_call(
        _tc_body,
        grid=grid,
        in_specs=[
            pl.BlockSpec((blk, dim_in), lambda i: (i, 0)),
            pl.BlockSpec((blk, dim_pe), lambda i: (i, 0)),
            pl.BlockSpec((dim_in, dim_h), lambda i: (0, 0)),
            pl.BlockSpec((1, dim_h), lambda i: (0, 0)),
        ],
        out_specs=pl.BlockSpec((blk, dim_out), lambda i: (i, 0)),
        out_shape=jax.ShapeDtypeStruct((n, dim_out), jnp.float32),
        compiler_params=pltpu.CompilerParams(
            dimension_semantics=("parallel",),
        ),
    )(x, pe, W, b2)


# speedup vs baseline: 1.0445x; 1.0226x over previous
import functools

import jax
from jax import lax
import jax.numpy as jnp
from jax.experimental import pallas as pl
from jax.experimental.pallas import tpu as pltpu
from jax.experimental.pallas import tpu_sc as plsc

_NC = 2
_NS = 16
_NW = _NC * _NS
_LANES = 16
_CHUNK = 800


def _sc_gather_body(spd0_hbm, spd1_hbm, tab2_hbm, pe_hbm,
                    idx0_v, idx1_v, pair_v, rows_v, sem_idx, sem_g, sem_st, *,
                    chunk_lo, chunk_hi, table_rows):
    cid = lax.axis_index("c")
    sid = lax.axis_index("s")
    wid = sid * _NC + cid

    base_c, extra = divmod(chunk_hi - chunk_lo, _NW)
    start = chunk_lo + wid * base_c + jnp.minimum(wid, extra)
    count = base_c + jnp.where(wid < extra, 1, 0)

    def fire_idx(ci, s):
        row0 = ci * _CHUNK
        pltpu.make_async_copy(spd0_hbm.at[pl.ds(row0, _CHUNK)],
                              idx0_v.at[s], sem_idx.at[s]).start()
        pltpu.make_async_copy(spd1_hbm.at[pl.ds(row0, _CHUNK)],
                              idx1_v.at[s], sem_idx.at[s]).start()

    def wait_idx(s):
        pltpu.make_async_copy(spd0_hbm.at[pl.ds(0, _CHUNK)],
                              idx0_v.at[s], sem_idx.at[s]).wait()
        pltpu.make_async_copy(spd1_hbm.at[pl.ds(0, _CHUNK)],
                              idx1_v.at[s], sem_idx.at[s]).wait()

    def chunk_body(ci, carry):
        r = ci - start
        s = lax.rem(r, 2)
        row_out = (ci - chunk_lo) * _CHUNK

        @pl.when(r == 0)
        def _():
            fire_idx(ci, s)

        wait_idx(s)

        def g(k, c2):
            a = idx0_v.at[s][pl.ds(k * _LANES, _LANES)]
            b = idx1_v.at[s][pl.ds(k * _LANES, _LANES)]
            pair_v[pl.ds(k * _LANES, _LANES)] = a * table_rows + b
            return c2

        lax.fori_loop(0, _CHUNK // _LANES, g, 0)

        @pl.when(r + 1 < count)
        def _():
            fire_idx(ci + 1, 1 - s)

        @pl.when(r >= 2)
        def _():
            pltpu.make_async_copy(rows_v.at[s],
                                  pe_hbm.at[pl.ds(row_out, _CHUNK)],
                                  sem_st.at[s]).wait()

        off = 0
        while off < _CHUNK:
            sz = min(128, _CHUNK - off)
            pltpu.async_copy(tab2_hbm.at[pair_v.at[pl.ds(off, sz)]],
                             rows_v.at[s, pl.ds(off, sz)], sem_g)
            off += sz
        pltpu.make_async_copy(pe_hbm.at[pl.ds(row_out, _CHUNK)],
                              rows_v.at[s], sem_g).wait()
        pltpu.make_async_copy(rows_v.at[s],
                              pe_hbm.at[pl.ds(row_out, _CHUNK)],
                              sem_st.at[s]).start()
        return carry

    lax.fori_loop(start, start + count, chunk_body, 0, unroll=False)

    @pl.when(count >= 2)
    def _():
        s = lax.rem(count - 2, 2)
        pltpu.make_async_copy(rows_v.at[s], pe_hbm.at[pl.ds(0, _CHUNK)],
                              sem_st.at[s]).wait()

    @pl.when(count >= 1)
    def _():
        s = lax.rem(count - 1, 2)
        pltpu.make_async_copy(rows_v.at[s], pe_hbm.at[pl.ds(0, _CHUNK)],
                              sem_st.at[s]).wait()


def _sc_gather(spd0, spd1, table2, chunk_lo, chunk_hi):
    table_rows = int(round(table2.shape[0] ** 0.5))
    dim_pe = table2.shape[1]
    seg_rows = (chunk_hi - chunk_lo) * _CHUNK
    mesh = plsc.VectorSubcoreMesh(core_axis_name="c", subcore_axis_name="s")
    k = functools.partial(
        pl.kernel,
        mesh=mesh,
        out_type=jax.ShapeDtypeStruct((seg_rows, dim_pe), jnp.float32),
        scratch_types=[
            pltpu.VMEM((2, _CHUNK), jnp.int32),
            pltpu.VMEM((2, _CHUNK), jnp.int32),
            pltpu.VMEM((_CHUNK,), jnp.int32),
            pltpu.VMEM((2, _CHUNK, dim_pe), jnp.float32),
            pltpu.SemaphoreType.DMA((2,)),
            pltpu.SemaphoreType.DMA,
            pltpu.SemaphoreType.DMA((2,)),
        ],
        compiler_params=pltpu.CompilerParams(use_tc_tiling_on_sc=False),
    )(functools.partial(_sc_gather_body, chunk_lo=chunk_lo,
                        chunk_hi=chunk_hi, table_rows=table_rows))
    return k(spd0, spd1, table2)


def _tc_body(x_ref, pe_ref, w_ref, b_ref, out_ref):
    h = jnp.dot(x_ref[...], w_ref[...], preferred_element_type=jnp.float32)
    out_ref[...] = jnp.concatenate([h + b_ref[...], pe_ref[...]], axis=1)


def _tc_body_chain(x_ref, pe_ref, w_ref, b_ref, prev_ref, out_ref):
    del prev_ref
    _tc_body(x_ref, pe_ref, w_ref, b_ref, out_ref)


def _tc_call(x, pe_seg, W, b2, prev, blk_lo, nblk, blk):
    n, dim_in = x.shape
    dim_h = W.shape[1]
    dim_pe = pe_seg.shape[1]
    dim_out = dim_h + dim_pe
    in_specs = [
        pl.BlockSpec((blk, dim_in), lambda i: (i + blk_lo, 0)),
        pl.BlockSpec((blk, dim_pe), lambda i: (i, 0)),
        pl.BlockSpec((dim_in, dim_h), lambda i: (0, 0)),
        pl.BlockSpec((1, dim_h), lambda i: (0, 0)),
    ]
    args = [x, pe_seg, W, b2]
    body = _tc_body
    aliases = {}
    if prev is not None:
        in_specs.append(pl.BlockSpec(memory_space=pl.ANY))
        args.append(prev)
        body = _tc_body_chain
        aliases = {4: 0}
    return pl.pallas_call(
        body,
        grid=(nblk,),
        in_specs=in_specs,
        out_specs=pl.BlockSpec((blk, dim_out), lambda i: (i + blk_lo, 0)),
        out_shape=jax.ShapeDtypeStruct((n, dim_out), jnp.float32),
        input_output_aliases=aliases,
        compiler_params=pltpu.CompilerParams(
            dimension_semantics=("parallel",),
        ),
    )(*args)


def kernel(x, spd, dist_table, W, b):
    n, dim_in = x.shape
    dim_h = W.shape[1]
    rows, half_pe = dist_table.shape

    table2 = jnp.concatenate(
        [jnp.repeat(dist_table, rows, axis=0), jnp.tile(dist_table, (rows, 1))],
        axis=1)
    spd0 = spd[:, 0]
    spd1 = spd[:, 1]
    b2 = b.reshape(1, dim_h)

    blk = 2000
    num_chunks = n // _CHUNK
    seg_chunk_bounds = (0, num_chunks // 2 // 5 * 5, num_chunks)

    out = None
    for lo, hi in zip(seg_chunk_bounds[:-1], seg_chunk_bounds[1:]):
        pe_seg = _sc_gather(spd0, spd1, table2, lo, hi)
        out = _tc_call(x, pe_seg, W, b2, out,
                       lo * _CHUNK // blk, (hi - lo) * _CHUNK // blk, blk)
    return out


# speedup vs baseline: 1.3383x; 1.2813x over previous
import functools

import jax
from jax import lax
import jax.numpy as jnp
from jax.experimental import pallas as pl
from jax.experimental.pallas import tpu as pltpu
from jax.experimental.pallas import tpu_sc as plsc

_NC = 2
_NS = 16
_NW = _NC * _NS
_LANES = 16
_CHUNK = 800


def _sc_gather_body(spd0_hbm, spd1_hbm, tab2_hbm, pe_hbm,
                    idx0_v, idx1_v, pair_v, rows_v, sem_idx, sem_g, sem_st, *,
                    chunk_lo, chunk_hi, table_rows):
    cid = lax.axis_index("c")
    sid = lax.axis_index("s")
    wid = sid * _NC + cid

    base_c, extra = divmod(chunk_hi - chunk_lo, _NW)
    start = chunk_lo + wid * base_c + jnp.minimum(wid, extra)
    count = base_c + jnp.where(wid < extra, 1, 0)

    def fire_idx(ci, s):
        row0 = ci * _CHUNK
        pltpu.make_async_copy(spd0_hbm.at[pl.ds(row0, _CHUNK)],
                              idx0_v.at[s], sem_idx.at[s]).start()
        pltpu.make_async_copy(spd1_hbm.at[pl.ds(row0, _CHUNK)],
                              idx1_v.at[s], sem_idx.at[s]).start()

    def wait_idx(s):
        pltpu.make_async_copy(spd0_hbm.at[pl.ds(0, _CHUNK)],
                              idx0_v.at[s], sem_idx.at[s]).wait()
        pltpu.make_async_copy(spd1_hbm.at[pl.ds(0, _CHUNK)],
                              idx1_v.at[s], sem_idx.at[s]).wait()

    def chunk_body(ci, carry):
        r = ci - start
        s = lax.rem(r, 2)
        row_out = (ci - chunk_lo) * _CHUNK

        @pl.when(r == 0)
        def _():
            fire_idx(ci, s)

        wait_idx(s)

        def g(k, c2):
            a = idx0_v.at[s][pl.ds(k * _LANES, _LANES)]
            b = idx1_v.at[s][pl.ds(k * _LANES, _LANES)]
            pair_v[pl.ds(k * _LANES, _LANES)] = a * table_rows + b
            return c2

        lax.fori_loop(0, _CHUNK // _LANES, g, 0)

        @pl.when(r + 1 < count)
        def _():
            fire_idx(ci + 1, 1 - s)

        @pl.when(r >= 2)
        def _():
            pltpu.make_async_copy(rows_v.at[s],
                                  pe_hbm.at[pl.ds(row_out, _CHUNK),
                                            pl.ds(0, rows_v.shape[2])],
                                  sem_st.at[s]).wait()

        off = 0
        while off < _CHUNK:
            sz = min(128, _CHUNK - off)
            pltpu.async_copy(tab2_hbm.at[pair_v.at[pl.ds(off, sz)]],
                             rows_v.at[s, pl.ds(off, sz)], sem_g)
            off += sz
        pltpu.make_async_copy(pe_hbm.at[pl.ds(row_out, _CHUNK),
                                        pl.ds(0, rows_v.shape[2])],
                              rows_v.at[s], sem_g).wait()
        pltpu.make_async_copy(rows_v.at[s],
                              pe_hbm.at[pl.ds(row_out, _CHUNK),
                                        pl.ds(0, rows_v.shape[2])],
                              sem_st.at[s]).start()
        return carry

    lax.fori_loop(start, start + count, chunk_body, 0, unroll=False)

    @pl.when(count >= 2)
    def _():
        s = lax.rem(count - 2, 2)
        pltpu.make_async_copy(rows_v.at[s],
                              pe_hbm.at[pl.ds(0, _CHUNK),
                                        pl.ds(0, rows_v.shape[2])],
                              sem_st.at[s]).wait()

    @pl.when(count >= 1)
    def _():
        s = lax.rem(count - 1, 2)
        pltpu.make_async_copy(rows_v.at[s],
                              pe_hbm.at[pl.ds(0, _CHUNK),
                                        pl.ds(0, rows_v.shape[2])],
                              sem_st.at[s]).wait()


def _sc_gather(spd0, spd1, table2, chunk_lo, chunk_hi):
    table_rows = int(round(table2.shape[0] ** 0.5))
    dim_pe = table2.shape[1]
    seg_rows = (chunk_hi - chunk_lo) * _CHUNK
    mesh = plsc.VectorSubcoreMesh(core_axis_name="c", subcore_axis_name="s")
    k = functools.partial(
        pl.kernel,
        mesh=mesh,
        out_type=jax.ShapeDtypeStruct((seg_rows, 2 * dim_pe), jnp.float32),
        scratch_types=[
            pltpu.VMEM((2, _CHUNK), jnp.int32),
            pltpu.VMEM((2, _CHUNK), jnp.int32),
            pltpu.VMEM((_CHUNK,), jnp.int32),
            pltpu.VMEM((2, _CHUNK, dim_pe), jnp.float32),
            pltpu.SemaphoreType.DMA((2,)),
            pltpu.SemaphoreType.DMA,
            pltpu.SemaphoreType.DMA((2,)),
        ],
        compiler_params=pltpu.CompilerParams(use_tc_tiling_on_sc=False),
    )(functools.partial(_sc_gather_body, chunk_lo=chunk_lo,
                        chunk_hi=chunk_hi, table_rows=table_rows))
    return k(spd0, spd1, table2)


def _tc_body(x_ref, pe_ref, w_ref, b_ref, out_ref):
    h = jnp.dot(x_ref[...], w_ref[...], preferred_element_type=jnp.float32)
    pe = pe_ref[...][:, : pe_ref.shape[1] // 2]
    out_ref[...] = jnp.concatenate([h + b_ref[...], pe], axis=1)


def _tc_body_chain(x_ref, pe_ref, w_ref, b_ref, prev_ref, out_ref):
    del prev_ref
    _tc_body(x_ref, pe_ref, w_ref, b_ref, out_ref)


def _tc_call(x, pe_seg, W, b2, prev, blk_lo, nblk, blk):
    n, dim_in = x.shape
    dim_h = W.shape[1]
    dim_pe = pe_seg.shape[1]
    dim_out = dim_h + dim_pe // 2
    in_specs = [
        pl.BlockSpec((blk, dim_in), lambda i: (i + blk_lo, 0)),
        pl.BlockSpec((blk, dim_pe), lambda i: (i, 0)),
        pl.BlockSpec((dim_in, dim_h), lambda i: (0, 0)),
        pl.BlockSpec((1, dim_h), lambda i: (0, 0)),
    ]
    args = [x, pe_seg, W, b2]
    body = _tc_body
    aliases = {}
    if prev is not None:
        in_specs.append(pl.BlockSpec(memory_space=pl.ANY))
        args.append(prev)
        body = _tc_body_chain
        aliases = {4: 0}
    return pl.pallas_call(
        body,
        grid=(nblk,),
        in_specs=in_specs,
        out_specs=pl.BlockSpec((blk, dim_out), lambda i: (i + blk_lo, 0)),
        out_shape=jax.ShapeDtypeStruct((n, dim_out), jnp.float32),
        input_output_aliases=aliases,
        compiler_params=pltpu.CompilerParams(
            dimension_semantics=("parallel",),
        ),
    )(*args)


def kernel(x, spd, dist_table, W, b):
    n, dim_in = x.shape
    dim_h = W.shape[1]
    rows, half_pe = dist_table.shape

    table2 = jnp.concatenate(
        [jnp.repeat(dist_table, rows, axis=0), jnp.tile(dist_table, (rows, 1))],
        axis=1)
    spd0 = spd[:, 0]
    spd1 = spd[:, 1]
    b2 = b.reshape(1, dim_h)

    blk = 2000
    num_chunks = n // _CHUNK
    seg_chunk_bounds = (0, num_chunks // 2 // 5 * 5, num_chunks)

    out = None
    for lo, hi in zip(seg_chunk_bounds[:-1], seg_chunk_bounds[1:]):
        pe_seg = _sc_gather(spd0, spd1, table2, lo, hi)
        out = _tc_call(x, pe_seg, W, b2, out,
                       lo * _CHUNK // blk, (hi - lo) * _CHUNK // blk, blk)
    return out


# speedup vs baseline: 1.3503x; 1.0090x over previous
import functools

import jax
from jax import lax
import jax.numpy as jnp
from jax.experimental import pallas as pl
from jax.experimental.pallas import tpu as pltpu
from jax.experimental.pallas import tpu_sc as plsc

_NC = 2
_NS = 16
_NW = _NC * _NS
_LANES = 16
_CHUNK = 800


def _sc_gather_body(spd0_hbm, spd1_hbm, tab2_hbm, pe_hbm,
                    idx0_v, idx1_v, pair_v, rows_v, sem_idx, sem_g, sem_st, *,
                    chunk_lo, chunk_hi, table_rows):
    cid = lax.axis_index("c")
    sid = lax.axis_index("s")
    wid = sid * _NC + cid

    base_c, extra = divmod(chunk_hi - chunk_lo, _NW)
    start = chunk_lo + wid * base_c + jnp.minimum(wid, extra)
    count = base_c + jnp.where(wid < extra, 1, 0)

    def fire_idx(ci, s):
        row0 = ci * _CHUNK
        pltpu.make_async_copy(spd0_hbm.at[pl.ds(row0, _CHUNK)],
                              idx0_v.at[s], sem_idx.at[s]).start()
        pltpu.make_async_copy(spd1_hbm.at[pl.ds(row0, _CHUNK)],
                              idx1_v.at[s], sem_idx.at[s]).start()

    def wait_idx(s):
        pltpu.make_async_copy(spd0_hbm.at[pl.ds(0, _CHUNK)],
                              idx0_v.at[s], sem_idx.at[s]).wait()
        pltpu.make_async_copy(spd1_hbm.at[pl.ds(0, _CHUNK)],
                              idx1_v.at[s], sem_idx.at[s]).wait()

    def chunk_body(ci, carry):
        r = ci - start
        s = lax.rem(r, 2)
        row_out = (ci - chunk_lo) * _CHUNK

        @pl.when(r == 0)
        def _():
            fire_idx(ci, s)

        wait_idx(s)

        def g(k, c2):
            a = idx0_v.at[s][pl.ds(k * _LANES, _LANES)]
            b = idx1_v.at[s][pl.ds(k * _LANES, _LANES)]
            pair_v[pl.ds(k * _LANES, _LANES)] = a * table_rows + b
            return c2

        lax.fori_loop(0, _CHUNK // _LANES, g, 0)

        @pl.when(r + 1 < count)
        def _():
            fire_idx(ci + 1, 1 - s)

        @pl.when(r >= 2)
        def _():
            pltpu.make_async_copy(rows_v.at[s],
                                  pe_hbm.at[pl.ds(row_out, _CHUNK),
                                            pl.ds(0, rows_v.shape[2])],
                                  sem_st.at[s]).wait()

        off = 0
        while off < _CHUNK:
            sz = min(128, _CHUNK - off)
            pltpu.async_copy(tab2_hbm.at[pair_v.at[pl.ds(off, sz)]],
                             rows_v.at[s, pl.ds(off, sz)], sem_g)
            off += sz
        pltpu.make_async_copy(pe_hbm.at[pl.ds(row_out, _CHUNK),
                                        pl.ds(0, rows_v.shape[2])],
                              rows_v.at[s], sem_g).wait()
        pltpu.make_async_copy(rows_v.at[s],
                              pe_hbm.at[pl.ds(row_out, _CHUNK),
                                        pl.ds(0, rows_v.shape[2])],
                              sem_st.at[s]).start()
        return carry

    lax.fori_loop(start, start + count, chunk_body, 0, unroll=False)

    @pl.when(count >= 2)
    def _():
        s = lax.rem(count - 2, 2)
        pltpu.make_async_copy(rows_v.at[s],
                              pe_hbm.at[pl.ds(0, _CHUNK),
                                        pl.ds(0, rows_v.shape[2])],
                              sem_st.at[s]).wait()

    @pl.when(count >= 1)
    def _():
        s = lax.rem(count - 1, 2)
        pltpu.make_async_copy(rows_v.at[s],
                              pe_hbm.at[pl.ds(0, _CHUNK),
                                        pl.ds(0, rows_v.shape[2])],
                              sem_st.at[s]).wait()


def _sc_gather(spd0, spd1, table2, chunk_lo, chunk_hi):
    table_rows = int(round(table2.shape[0] ** 0.5))
    dim_pe = table2.shape[1]
    seg_rows = (chunk_hi - chunk_lo) * _CHUNK
    mesh = plsc.VectorSubcoreMesh(core_axis_name="c", subcore_axis_name="s")
    k = functools.partial(
        pl.kernel,
        mesh=mesh,
        out_type=jax.ShapeDtypeStruct((seg_rows, 2 * dim_pe), jnp.float32),
        scratch_types=[
            pltpu.VMEM((2, _CHUNK), jnp.int32),
            pltpu.VMEM((2, _CHUNK), jnp.int32),
            pltpu.VMEM((_CHUNK,), jnp.int32),
            pltpu.VMEM((2, _CHUNK, dim_pe), jnp.float32),
            pltpu.SemaphoreType.DMA((2,)),
            pltpu.SemaphoreType.DMA,
            pltpu.SemaphoreType.DMA((2,)),
        ],
        compiler_params=pltpu.CompilerParams(use_tc_tiling_on_sc=False),
    )(functools.partial(_sc_gather_body, chunk_lo=chunk_lo,
                        chunk_hi=chunk_hi, table_rows=table_rows))
    return k(spd0, spd1, table2)


def _tc_body(x_ref, pe_ref, w_ref, b_ref, out_ref):
    h = jnp.dot(x_ref[...], w_ref[...], preferred_element_type=jnp.float32)
    pe = pe_ref[...][:, : pe_ref.shape[1] // 2]
    out_ref[...] = jnp.concatenate([h + b_ref[...], pe], axis=1)


def _tc_body_chain(x_ref, pe_ref, w_ref, b_ref, prev_ref, out_ref):
    del prev_ref
    _tc_body(x_ref, pe_ref, w_ref, b_ref, out_ref)


def _tc_call(x, pe_seg, W, b2, prev, blk_lo, nblk, blk):
    n, dim_in = x.shape
    dim_h = W.shape[1]
    dim_pe = pe_seg.shape[1]
    dim_out = dim_h + dim_pe // 2
    in_specs = [
        pl.BlockSpec((blk, dim_in), lambda i: (i + blk_lo, 0)),
        pl.BlockSpec((blk, dim_pe), lambda i: (i, 0)),
        pl.BlockSpec((dim_in, dim_h), lambda i: (0, 0)),
        pl.BlockSpec((1, dim_h), lambda i: (0, 0)),
    ]
    args = [x, pe_seg, W, b2]
    body = _tc_body
    aliases = {}
    if prev is not None:
        in_specs.append(pl.BlockSpec(memory_space=pl.ANY))
        args.append(prev)
        body = _tc_body_chain
        aliases = {4: 0}
    return pl.pallas_call(
        body,
        grid=(nblk,),
        in_specs=in_specs,
        out_specs=pl.BlockSpec((blk, dim_out), lambda i: (i + blk_lo, 0)),
        out_shape=jax.ShapeDtypeStruct((n, dim_out), jnp.float32),
        input_output_aliases=aliases,
        compiler_params=pltpu.CompilerParams(
            dimension_semantics=("parallel",),
        ),
    )(*args)


def kernel(x, spd, dist_table, W, b):
    n, dim_in = x.shape
    dim_h = W.shape[1]
    rows, half_pe = dist_table.shape

    table2 = jnp.concatenate(
        [jnp.repeat(dist_table, rows, axis=0), jnp.tile(dist_table, (rows, 1))],
        axis=1)
    spd0 = spd[:, 0]
    spd1 = spd[:, 1]
    b2 = b.reshape(1, dim_h)

    blk = 2000
    num_chunks = n // _CHUNK
    seg_chunk_bounds = (0, num_chunks * 2 // 5 // 5 * 5, num_chunks)

    out = None
    for lo, hi in zip(seg_chunk_bounds[:-1], seg_chunk_bounds[1:]):
        pe_seg = _sc_gather(spd0, spd1, table2, lo, hi)
        out = _tc_call(x, pe_seg, W, b2, out,
                       lo * _CHUNK // blk, (hi - lo) * _CHUNK // blk, blk)
    return out


# speedup vs baseline: 1.4127x; 1.0462x over previous
import functools

import jax
from jax import lax
import jax.numpy as jnp
from jax.experimental import pallas as pl
from jax.experimental.pallas import tpu as pltpu
from jax.experimental.pallas import tpu_sc as plsc

_NC = 2
_NS = 16
_NW = _NC * _NS
_LANES = 16
_CHUNK = 800


def _sc_gather_body(spd0_hbm, spd1_hbm, tab2_hbm, pe_hbm,
                    idx0_v, idx1_v, pair_v, rows_v, sem_idx, sem_g, sem_st, *,
                    chunk_lo, chunk_hi, table_rows):
    cid = lax.axis_index("c")
    sid = lax.axis_index("s")
    wid = sid * _NC + cid

    base_c, extra = divmod(chunk_hi - chunk_lo, _NW)
    start = chunk_lo + wid * base_c + jnp.minimum(wid, extra)
    count = base_c + jnp.where(wid < extra, 1, 0)

    def fire_idx(ci, s):
        row0 = ci * _CHUNK
        pltpu.make_async_copy(spd0_hbm.at[pl.ds(row0, _CHUNK)],
                              idx0_v.at[s], sem_idx.at[s]).start()
        pltpu.make_async_copy(spd1_hbm.at[pl.ds(row0, _CHUNK)],
                              idx1_v.at[s], sem_idx.at[s]).start()

    def wait_idx(s):
        pltpu.make_async_copy(spd0_hbm.at[pl.ds(0, _CHUNK)],
                              idx0_v.at[s], sem_idx.at[s]).wait()
        pltpu.make_async_copy(spd1_hbm.at[pl.ds(0, _CHUNK)],
                              idx1_v.at[s], sem_idx.at[s]).wait()

    def chunk_body(ci, carry):
        r = ci - start
        s = lax.rem(r, 2)
        row_out = (ci - chunk_lo) * _CHUNK

        @pl.when(r == 0)
        def _():
            fire_idx(ci, s)

        wait_idx(s)

        def g(k, c2):
            a = idx0_v.at[s][pl.ds(k * _LANES, _LANES)]
            b = idx1_v.at[s][pl.ds(k * _LANES, _LANES)]
            pair_v[pl.ds(k * _LANES, _LANES)] = a * table_rows + b
            return c2

        lax.fori_loop(0, _CHUNK // _LANES, g, 0)

        @pl.when(r + 1 < count)
        def _():
            fire_idx(ci + 1, 1 - s)

        @pl.when(r >= 2)
        def _():
            pltpu.make_async_copy(rows_v.at[s],
                                  pe_hbm.at[pl.ds(row_out, _CHUNK),
                                            pl.ds(0, rows_v.shape[2])],
                                  sem_st.at[s]).wait()

        off = 0
        while off < _CHUNK:
            sz = min(128, _CHUNK - off)
            pltpu.async_copy(tab2_hbm.at[pair_v.at[pl.ds(off, sz)]],
                             rows_v.at[s, pl.ds(off, sz)], sem_g)
            off += sz
        pltpu.make_async_copy(pe_hbm.at[pl.ds(row_out, _CHUNK),
                                        pl.ds(0, rows_v.shape[2])],
                              rows_v.at[s], sem_g).wait()
        pltpu.make_async_copy(rows_v.at[s],
                              pe_hbm.at[pl.ds(row_out, _CHUNK),
                                        pl.ds(0, rows_v.shape[2])],
                              sem_st.at[s]).start()
        return carry

    lax.fori_loop(start, start + count, chunk_body, 0, unroll=False)

    @pl.when(count >= 2)
    def _():
        s = lax.rem(count - 2, 2)
        pltpu.make_async_copy(rows_v.at[s],
                              pe_hbm.at[pl.ds(0, _CHUNK),
                                        pl.ds(0, rows_v.shape[2])],
                              sem_st.at[s]).wait()

    @pl.when(count >= 1)
    def _():
        s = lax.rem(count - 1, 2)
        pltpu.make_async_copy(rows_v.at[s],
                              pe_hbm.at[pl.ds(0, _CHUNK),
                                        pl.ds(0, rows_v.shape[2])],
                              sem_st.at[s]).wait()


def _sc_gather(spd0, spd1, table2, chunk_lo, chunk_hi):
    table_rows = int(round(table2.shape[0] ** 0.5))
    dim_pe = table2.shape[1]
    seg_rows = (chunk_hi - chunk_lo) * _CHUNK
    mesh = plsc.VectorSubcoreMesh(core_axis_name="c", subcore_axis_name="s")
    k = functools.partial(
        pl.kernel,
        mesh=mesh,
        out_type=jax.ShapeDtypeStruct((seg_rows, 2 * dim_pe), jnp.float32),
        scratch_types=[
            pltpu.VMEM((2, _CHUNK), jnp.int32),
            pltpu.VMEM((2, _CHUNK), jnp.int32),
            pltpu.VMEM((_CHUNK,), jnp.int32),
            pltpu.VMEM((2, _CHUNK, dim_pe), jnp.float32),
            pltpu.SemaphoreType.DMA((2,)),
            pltpu.SemaphoreType.DMA,
            pltpu.SemaphoreType.DMA((2,)),
        ],
        compiler_params=pltpu.CompilerParams(use_tc_tiling_on_sc=False),
    )(functools.partial(_sc_gather_body, chunk_lo=chunk_lo,
                        chunk_hi=chunk_hi, table_rows=table_rows))
    return k(spd0, spd1, table2)


def _tc_body(x_ref, pe_ref, w_ref, b_ref, out_ref):
    h = jnp.dot(x_ref[...], w_ref[...], preferred_element_type=jnp.float32)
    pe = pe_ref[...][:, : pe_ref.shape[1] // 2]
    out_ref[...] = jnp.concatenate([h + b_ref[...], pe], axis=1)


def _tc_body_chain(x_ref, pe_ref, w_ref, b_ref, prev_ref, out_ref):
    del prev_ref
    _tc_body(x_ref, pe_ref, w_ref, b_ref, out_ref)


def _tc_call(x, pe_seg, W, b2, prev, blk_lo, nblk, blk):
    n, dim_in = x.shape
    dim_h = W.shape[1]
    dim_pe = pe_seg.shape[1]
    dim_out = dim_h + dim_pe // 2
    in_specs = [
        pl.BlockSpec((blk, dim_in), lambda i: (i + blk_lo, 0)),
        pl.BlockSpec((blk, dim_pe), lambda i: (i, 0)),
        pl.BlockSpec((dim_in, dim_h), lambda i: (0, 0)),
        pl.BlockSpec((1, dim_h), lambda i: (0, 0)),
    ]
    args = [x, pe_seg, W, b2]
    body = _tc_body
    aliases = {}
    if prev is not None:
        in_specs.append(pl.BlockSpec(memory_space=pl.ANY))
        args.append(prev)
        body = _tc_body_chain
        aliases = {4: 0}
    return pl.pallas_call(
        body,
        grid=(nblk,),
        in_specs=in_specs,
        out_specs=pl.BlockSpec((blk, dim_out), lambda i: (i + blk_lo, 0)),
        out_shape=jax.ShapeDtypeStruct((n, dim_out), jnp.float32),
        input_output_aliases=aliases,
        compiler_params=pltpu.CompilerParams(
            dimension_semantics=("parallel",),
        ),
    )(*args)


def kernel(x, spd, dist_table, W, b):
    n, dim_in = x.shape
    dim_h = W.shape[1]
    rows, half_pe = dist_table.shape

    table2 = jnp.concatenate(
        [jnp.repeat(dist_table, rows, axis=0), jnp.tile(dist_table, (rows, 1))],
        axis=1)
    spd0 = spd[:, 0]
    spd1 = spd[:, 1]
    b2 = b.reshape(1, dim_h)

    blk = 4000
    num_chunks = n // _CHUNK
    seg_chunk_bounds = (0, num_chunks * 2 // 5 // 5 * 5, num_chunks)

    out = None
    for lo, hi in zip(seg_chunk_bounds[:-1], seg_chunk_bounds[1:]):
        pe_seg = _sc_gather(spd0, spd1, table2, lo, hi)
        out = _tc_call(x, pe_seg, W, b2, out,
                       lo * _CHUNK // blk, (hi - lo) * _CHUNK // blk, blk)
    return out


# speedup vs baseline: 1.4413x; 1.0202x over previous
import functools

import jax
from jax import lax
import jax.numpy as jnp
from jax.experimental import pallas as pl
from jax.experimental.pallas import tpu as pltpu
from jax.experimental.pallas import tpu_sc as plsc

_NC = 2
_NS = 16
_NW = _NC * _NS
_LANES = 16
_CHUNK = 800


def _sc_gather_body(spd0_hbm, spd1_hbm, tab2_hbm, pe_hbm,
                    idx0_v, idx1_v, pair_v, rows_v, sem_idx, sem_g, sem_st, *,
                    chunk_lo, chunk_hi, table_rows):
    cid = lax.axis_index("c")
    sid = lax.axis_index("s")
    wid = sid * _NC + cid

    base_c, extra = divmod(chunk_hi - chunk_lo, _NW)
    start = chunk_lo + wid * base_c + jnp.minimum(wid, extra)
    count = base_c + jnp.where(wid < extra, 1, 0)

    def fire_idx(ci, s):
        row0 = ci * _CHUNK
        pltpu.make_async_copy(spd0_hbm.at[pl.ds(row0, _CHUNK)],
                              idx0_v.at[s], sem_idx.at[s]).start()
        pltpu.make_async_copy(spd1_hbm.at[pl.ds(row0, _CHUNK)],
                              idx1_v.at[s], sem_idx.at[s]).start()

    def wait_idx(s):
        pltpu.make_async_copy(spd0_hbm.at[pl.ds(0, _CHUNK)],
                              idx0_v.at[s], sem_idx.at[s]).wait()
        pltpu.make_async_copy(spd1_hbm.at[pl.ds(0, _CHUNK)],
                              idx1_v.at[s], sem_idx.at[s]).wait()

    def chunk_body(ci, carry):
        r = ci - start
        s = lax.rem(r, 2)
        row_out = (ci - chunk_lo) * _CHUNK

        @pl.when(r == 0)
        def _():
            fire_idx(ci, s)

        wait_idx(s)

        def g(k, c2):
            a = idx0_v.at[s][pl.ds(k * _LANES, _LANES)]
            b = idx1_v.at[s][pl.ds(k * _LANES, _LANES)]
            pair_v[pl.ds(k * _LANES, _LANES)] = a * table_rows + b
            return c2

        lax.fori_loop(0, _CHUNK // _LANES, g, 0)

        @pl.when(r + 1 < count)
        def _():
            fire_idx(ci + 1, 1 - s)

        @pl.when(r >= 2)
        def _():
            pltpu.make_async_copy(rows_v.at[s],
                                  pe_hbm.at[pl.ds(row_out, _CHUNK),
                                            pl.ds(0, rows_v.shape[2])],
                                  sem_st.at[s]).wait()

        off = 0
        while off < _CHUNK:
            sz = min(128, _CHUNK - off)
            pltpu.async_copy(tab2_hbm.at[pair_v.at[pl.ds(off, sz)]],
                             rows_v.at[s, pl.ds(off, sz)], sem_g)
            off += sz
        pltpu.make_async_copy(pe_hbm.at[pl.ds(row_out, _CHUNK),
                                        pl.ds(0, rows_v.shape[2])],
                              rows_v.at[s], sem_g).wait()
        pltpu.make_async_copy(rows_v.at[s],
                              pe_hbm.at[pl.ds(row_out, _CHUNK),
                                        pl.ds(0, rows_v.shape[2])],
                              sem_st.at[s]).start()
        return carry

    lax.fori_loop(start, start + count, chunk_body, 0, unroll=False)

    @pl.when(count >= 2)
    def _():
        s = lax.rem(count - 2, 2)
        pltpu.make_async_copy(rows_v.at[s],
                              pe_hbm.at[pl.ds(0, _CHUNK),
                                        pl.ds(0, rows_v.shape[2])],
                              sem_st.at[s]).wait()

    @pl.when(count >= 1)
    def _():
        s = lax.rem(count - 1, 2)
        pltpu.make_async_copy(rows_v.at[s],
                              pe_hbm.at[pl.ds(0, _CHUNK),
                                        pl.ds(0, rows_v.shape[2])],
                              sem_st.at[s]).wait()


def _sc_gather(spd0, spd1, table2, chunk_lo, chunk_hi):
    table_rows = int(round(table2.shape[0] ** 0.5))
    dim_pe = table2.shape[1]
    seg_rows = (chunk_hi - chunk_lo) * _CHUNK
    mesh = plsc.VectorSubcoreMesh(core_axis_name="c", subcore_axis_name="s")
    k = functools.partial(
        pl.kernel,
        mesh=mesh,
        out_type=jax.ShapeDtypeStruct((seg_rows, 2 * dim_pe), jnp.float32),
        scratch_types=[
            pltpu.VMEM((2, _CHUNK), jnp.int32),
            pltpu.VMEM((2, _CHUNK), jnp.int32),
            pltpu.VMEM((_CHUNK,), jnp.int32),
            pltpu.VMEM((2, _CHUNK, dim_pe), jnp.float32),
            pltpu.SemaphoreType.DMA((2,)),
            pltpu.SemaphoreType.DMA,
            pltpu.SemaphoreType.DMA((2,)),
        ],
        compiler_params=pltpu.CompilerParams(use_tc_tiling_on_sc=False),
    )(functools.partial(_sc_gather_body, chunk_lo=chunk_lo,
                        chunk_hi=chunk_hi, table_rows=table_rows))
    return k(spd0, spd1, table2)


def _tc_body(x_ref, pe_ref, w_ref, b_ref, out_ref):
    h = jnp.dot(x_ref[...], w_ref[...], preferred_element_type=jnp.float32)
    pe = pe_ref[...][:, : pe_ref.shape[1] // 2]
    out_ref[...] = jnp.concatenate([h + b_ref[...], pe], axis=1)


def _tc_body_chain(x_ref, pe_ref, w_ref, b_ref, prev_ref, out_ref):
    del prev_ref
    _tc_body(x_ref, pe_ref, w_ref, b_ref, out_ref)


def _tc_call(x, pe_seg, W, b2, prev, blk_lo, nblk, blk):
    n, dim_in = x.shape
    dim_h = W.shape[1]
    dim_pe = pe_seg.shape[1]
    dim_out = dim_h + dim_pe // 2
    in_specs = [
        pl.BlockSpec((blk, dim_in), lambda i: (i + blk_lo, 0)),
        pl.BlockSpec((blk, dim_pe), lambda i: (i, 0)),
        pl.BlockSpec((dim_in, dim_h), lambda i: (0, 0)),
        pl.BlockSpec((1, dim_h), lambda i: (0, 0)),
    ]
    args = [x, pe_seg, W, b2]
    body = _tc_body
    aliases = {}
    if prev is not None:
        in_specs.append(pl.BlockSpec(memory_space=pl.ANY))
        args.append(prev)
        body = _tc_body_chain
        aliases = {4: 0}
    return pl.pallas_call(
        body,
        grid=(nblk,),
        in_specs=in_specs,
        out_specs=pl.BlockSpec((blk, dim_out), lambda i: (i + blk_lo, 0)),
        out_shape=jax.ShapeDtypeStruct((n, dim_out), jnp.float32),
        input_output_aliases=aliases,
        compiler_params=pltpu.CompilerParams(
            dimension_semantics=("parallel",),
        ),
    )(*args)


def kernel(x, spd, dist_table, W, b):
    n, dim_in = x.shape
    dim_h = W.shape[1]
    rows, half_pe = dist_table.shape

    table2 = jnp.concatenate(
        [jnp.repeat(dist_table, rows, axis=0), jnp.tile(dist_table, (rows, 1))],
        axis=1)
    spd0 = spd[:, 0]
    spd1 = spd[:, 1]
    b2 = b.reshape(1, dim_h)

    blk = 5000
    num_chunks = n // _CHUNK
    seg_chunk_bounds = (0, num_chunks * 2 // 5 // 5 * 5, num_chunks)

    out = None
    for lo, hi in zip(seg_chunk_bounds[:-1], seg_chunk_bounds[1:]):
        pe_seg = _sc_gather(spd0, spd1, table2, lo, hi)
        out = _tc_call(x, pe_seg, W, b2, out,
                       lo * _CHUNK // blk, (hi - lo) * _CHUNK // blk, blk)
    return out


# speedup vs baseline: 1.4481x; 1.0048x over previous
import functools

import jax
from jax import lax
import jax.numpy as jnp
from jax.experimental import pallas as pl
from jax.experimental.pallas import tpu as pltpu
from jax.experimental.pallas import tpu_sc as plsc

_NC = 2
_NS = 16
_NW = _NC * _NS
_LANES = 16
_CHUNK = 800


def _sc_gather_body(spd0_hbm, spd1_hbm, tab2_hbm, pe_hbm,
                    idx0_v, idx1_v, pair_v, rows_v, sem_idx, sem_g, sem_st, *,
                    chunk_lo, chunk_hi, table_rows):
    cid = lax.axis_index("c")
    sid = lax.axis_index("s")
    wid = sid * _NC + cid

    base_c, extra = divmod(chunk_hi - chunk_lo, _NW)
    start = chunk_lo + wid * base_c + jnp.minimum(wid, extra)
    count = base_c + jnp.where(wid < extra, 1, 0)

    def fire_idx(ci, s):
        row0 = ci * _CHUNK
        pltpu.make_async_copy(spd0_hbm.at[pl.ds(row0, _CHUNK)],
                              idx0_v.at[s], sem_idx.at[s]).start()
        pltpu.make_async_copy(spd1_hbm.at[pl.ds(row0, _CHUNK)],
                              idx1_v.at[s], sem_idx.at[s]).start()

    def wait_idx(s):
        pltpu.make_async_copy(spd0_hbm.at[pl.ds(0, _CHUNK)],
                              idx0_v.at[s], sem_idx.at[s]).wait()
        pltpu.make_async_copy(spd1_hbm.at[pl.ds(0, _CHUNK)],
                              idx1_v.at[s], sem_idx.at[s]).wait()

    def chunk_body(ci, carry):
        r = ci - start
        s = lax.rem(r, 2)
        row_out = (ci - chunk_lo) * _CHUNK

        @pl.when(r == 0)
        def _():
            fire_idx(ci, s)

        wait_idx(s)

        def g(k, c2):
            a = idx0_v.at[s][pl.ds(k * _LANES, _LANES)]
            b = idx1_v.at[s][pl.ds(k * _LANES, _LANES)]
            pair_v[pl.ds(k * _LANES, _LANES)] = a * table_rows + b
            return c2

        lax.fori_loop(0, _CHUNK // _LANES, g, 0)

        @pl.when(r + 1 < count)
        def _():
            fire_idx(ci + 1, 1 - s)

        @pl.when(r >= 2)
        def _():
            pltpu.make_async_copy(rows_v.at[s],
                                  pe_hbm.at[pl.ds(row_out, _CHUNK),
                                            pl.ds(0, rows_v.shape[2])],
                                  sem_st.at[s]).wait()

        off = 0
        while off < _CHUNK:
            sz = min(128, _CHUNK - off)
            pltpu.async_copy(tab2_hbm.at[pair_v.at[pl.ds(off, sz)]],
                             rows_v.at[s, pl.ds(off, sz)], sem_g)
            off += sz
        pltpu.make_async_copy(pe_hbm.at[pl.ds(row_out, _CHUNK),
                                        pl.ds(0, rows_v.shape[2])],
                              rows_v.at[s], sem_g).wait()
        pltpu.make_async_copy(rows_v.at[s],
                              pe_hbm.at[pl.ds(row_out, _CHUNK),
                                        pl.ds(0, rows_v.shape[2])],
                              sem_st.at[s]).start()
        return carry

    lax.fori_loop(start, start + count, chunk_body, 0, unroll=False)

    @pl.when(count >= 2)
    def _():
        s = lax.rem(count - 2, 2)
        pltpu.make_async_copy(rows_v.at[s],
                              pe_hbm.at[pl.ds(0, _CHUNK),
                                        pl.ds(0, rows_v.shape[2])],
                              sem_st.at[s]).wait()

    @pl.when(count >= 1)
    def _():
        s = lax.rem(count - 1, 2)
        pltpu.make_async_copy(rows_v.at[s],
                              pe_hbm.at[pl.ds(0, _CHUNK),
                                        pl.ds(0, rows_v.shape[2])],
                              sem_st.at[s]).wait()


def _sc_gather(spd0, spd1, table2, chunk_lo, chunk_hi):
    table_rows = int(round(table2.shape[0] ** 0.5))
    dim_pe = table2.shape[1]
    seg_rows = (chunk_hi - chunk_lo) * _CHUNK
    mesh = plsc.VectorSubcoreMesh(core_axis_name="c", subcore_axis_name="s")
    k = functools.partial(
        pl.kernel,
        mesh=mesh,
        out_type=jax.ShapeDtypeStruct((seg_rows, 2 * dim_pe), jnp.float32),
        scratch_types=[
            pltpu.VMEM((2, _CHUNK), jnp.int32),
            pltpu.VMEM((2, _CHUNK), jnp.int32),
            pltpu.VMEM((_CHUNK,), jnp.int32),
            pltpu.VMEM((2, _CHUNK, dim_pe), jnp.float32),
            pltpu.SemaphoreType.DMA((2,)),
            pltpu.SemaphoreType.DMA,
            pltpu.SemaphoreType.DMA((2,)),
        ],
        compiler_params=pltpu.CompilerParams(use_tc_tiling_on_sc=False),
    )(functools.partial(_sc_gather_body, chunk_lo=chunk_lo,
                        chunk_hi=chunk_hi, table_rows=table_rows))
    return k(spd0, spd1, table2)


def _tc_body(x_ref, pe_ref, w_ref, b_ref, out_ref):
    h = jnp.dot(x_ref[...], w_ref[...], preferred_element_type=jnp.float32)
    pe = pe_ref[...][:, : pe_ref.shape[1] // 2]
    out_ref[...] = jnp.concatenate([h + b_ref[...], pe], axis=1)


def _tc_body_chain(x_ref, pe_ref, w_ref, b_ref, prev_ref, out_ref):
    del prev_ref
    _tc_body(x_ref, pe_ref, w_ref, b_ref, out_ref)


def _tc_call(x, pe_seg, W, b2, prev, blk_lo, nblk, blk):
    n, dim_in = x.shape
    dim_h = W.shape[1]
    dim_pe = pe_seg.shape[1]
    dim_out = dim_h + dim_pe // 2
    in_specs = [
        pl.BlockSpec((blk, dim_in), lambda i: (i + blk_lo, 0)),
        pl.BlockSpec((blk, dim_pe), lambda i: (i, 0)),
        pl.BlockSpec((dim_in, dim_h), lambda i: (0, 0)),
        pl.BlockSpec((1, dim_h), lambda i: (0, 0)),
    ]
    args = [x, pe_seg, W, b2]
    body = _tc_body
    aliases = {}
    if prev is not None:
        in_specs.append(pl.BlockSpec(memory_space=pl.ANY))
        args.append(prev)
        body = _tc_body_chain
        aliases = {4: 0}
    return pl.pallas_call(
        body,
        grid=(nblk,),
        in_specs=in_specs,
        out_specs=pl.BlockSpec((blk, dim_out), lambda i: (i + blk_lo, 0)),
        out_shape=jax.ShapeDtypeStruct((n, dim_out), jnp.float32),
        input_output_aliases=aliases,
        compiler_params=pltpu.CompilerParams(
            dimension_semantics=("parallel",),
        ),
    )(*args)


def kernel(x, spd, dist_table, W, b):
    n, dim_in = x.shape
    dim_h = W.shape[1]
    rows, half_pe = dist_table.shape

    table2 = jnp.concatenate(
        [jnp.repeat(dist_table, rows, axis=0), jnp.tile(dist_table, (rows, 1))],
        axis=1)
    spd0 = spd[:, 0]
    spd1 = spd[:, 1]
    b2 = b.reshape(1, dim_h)

    blk = 10000
    num_chunks = n // _CHUNK
    seg_chunk_bounds = (0, num_chunks * 2 // 5 // 5 * 5, num_chunks)

    out = None
    for lo, hi in zip(seg_chunk_bounds[:-1], seg_chunk_bounds[1:]):
        pe_seg = _sc_gather(spd0, spd1, table2, lo, hi)
        out = _tc_call(x, pe_seg, W, b2, out,
                       lo * _CHUNK // blk, (hi - lo) * _CHUNK // blk, blk)
    return out
